# baseline (speedup 1.0000x reference)
import jax
import jax.numpy as jnp
from jax import lax
from jax.experimental import pallas as pl
from jax.experimental.pallas import tpu as pltpu
from jax.experimental.pallas import tpu_sc as plsc

N_CB = 256
D = 10000
B = 4096
L = 16
NC = 2
NS = 16
NW = NC * NS
BPW = B // NW
DPAD = 10112
NBUF = 2
UNROLL = 25


def _body(mem_hbm, item_hbm, pos_hbm, gate_hbm, codes_hbm, out_hbm,
          idx_v, gate_v, code_v, mem_v, item_v, out_v, c_sem, m_sem, i_sem, o_sem):
    wid = lax.axis_index("s") * NC + lax.axis_index("c")
    base = wid * BPW

    pltpu.sync_copy(pos_hbm.at[pl.ds(base * 8, BPW * 8)], idx_v)
    pltpu.sync_copy(gate_hbm.at[pl.ds(base * L, BPW * L)], gate_v)

    ncb = jnp.full((L,), N_CB, jnp.int32)
    for i in range(BPW * 8 // L):
        s = pl.ds(i * L, L)
        idx_v[s] = lax.rem(idx_v[s], ncb)

    def issue_in(r, b):
        pltpu.async_copy(
            codes_hbm.at[idx_v.at[pl.ds(pl.multiple_of(r * 8, 8), 1)]],
            code_v.at[b], c_sem.at[b])
        pltpu.async_copy(mem_hbm.at[pl.ds(base + r, 1)], mem_v.at[b], m_sem.at[b])
        pltpu.async_copy(item_hbm.at[pl.ds(base + r, 1)], item_v.at[b], i_sem.at[b])

    for b in range(NBUF):
        issue_in(b, b)

    def step(g, carry):
        for b in range(NBUF):
            r = g * NBUF + b
            pltpu.make_async_copy(codes_hbm.at[pl.ds(0, 1)], code_v.at[b], c_sem.at[b]).wait()
            pltpu.make_async_copy(mem_hbm.at[pl.ds(0, 1)], mem_v.at[b], m_sem.at[b]).wait()
            pltpu.make_async_copy(item_hbm.at[pl.ds(0, 1)], item_v.at[b], i_sem.at[b]).wait()

            @pl.when(g > 0)
            def _wait_out():
                pltpu.make_async_copy(out_v.at[b], out_hbm.at[pl.ds(0, 1)], o_sem.at[b]).wait()

            gv = gate_v[pl.ds(pl.multiple_of(r * L, L), L)]
            one_m_g = 1.0 - gv

            def col(j, c2):
                for u in range(UNROLL):
                    cs = pl.ds(j * (L * UNROLL) + u * L, L)
                    out_v.at[b][0, cs] = (
                        gv * mem_v.at[b][0, cs]
                        + one_m_g * (item_v.at[b][0, cs] * code_v.at[b][0, cs]))
                return c2

            lax.fori_loop(0, D // L // UNROLL, col, 0)

            pltpu.async_copy(out_v.at[b], out_hbm.at[pl.ds(base + r, 1)], o_sem.at[b])

            @pl.when(r + NBUF < BPW)
            def _prefetch():
                issue_in(r + NBUF, b)
        return carry

    lax.fori_loop(0, BPW // NBUF, step, 0)

    for b in range(NBUF):
        pltpu.make_async_copy(out_v.at[b], out_hbm.at[pl.ds(0, 1)], o_sem.at[b]).wait()


@jax.jit
def kernel(memory, item, position, gate, position_codes):
    pos8 = jnp.repeat(position.astype(jnp.int32), 8)
    gate16 = jnp.repeat(gate.reshape(-1), L)
    codes_pad = jnp.pad(position_codes, ((0, 0), (0, DPAD - D)))
    f = pl.kernel(
        _body,
        out_type=jax.ShapeDtypeStruct((B, D), jnp.float32),
        mesh=plsc.VectorSubcoreMesh(core_axis_name="c", subcore_axis_name="s"),
        scratch_types=[
            pltpu.VMEM((BPW * 8,), jnp.int32),
            pltpu.VMEM((BPW * L,), jnp.float32),
            pltpu.VMEM((NBUF, 1, DPAD), jnp.float32),
            pltpu.VMEM((NBUF, 1, D), jnp.float32),
            pltpu.VMEM((NBUF, 1, D), jnp.float32),
            pltpu.VMEM((NBUF, 1, D), jnp.float32),
            pltpu.SemaphoreType.DMA((NBUF,)),
            pltpu.SemaphoreType.DMA((NBUF,)),
            pltpu.SemaphoreType.DMA((NBUF,)),
            pltpu.SemaphoreType.DMA((NBUF,)),
        ],
    )
    return f(memory, item, pos8, gate16, codes_pad)


# speedup vs baseline: 1.5200x; 1.5200x over previous
import jax
import jax.numpy as jnp
from jax import lax
from jax.experimental import pallas as pl
from jax.experimental.pallas import tpu as pltpu
from jax.experimental.pallas import tpu_sc as plsc

N_CB = 256
D = 10000
B = 4096
L = 16
NC = 2
NS = 16
NW = NC * NS
BPW = B // NW
DPAD = 10112
NBUF = 2
UNROLL = 5


def _body(mem_hbm, item_hbm, pos_hbm, gate_hbm, codes_hbm, out_hbm,
          idx_v, gate_v, code_v, mem_v, item_v, out_v, c_sem, m_sem, i_sem, o_sem):
    wid = lax.axis_index("s") * NC + lax.axis_index("c")
    base = wid * BPW

    pltpu.sync_copy(pos_hbm.at[pl.ds(base * 8, BPW * 8)], idx_v)
    pltpu.sync_copy(gate_hbm.at[pl.ds(base * L, BPW * L)], gate_v)

    ncb = jnp.full((L,), N_CB, jnp.int32)
    for i in range(BPW * 8 // L):
        s = pl.ds(i * L, L)
        idx_v[s] = lax.rem(idx_v[s], ncb)

    def issue_in(r, b):
        pltpu.async_copy(
            codes_hbm.at[idx_v.at[pl.ds(pl.multiple_of(r * 8, 8), 1)]],
            code_v.at[b], c_sem.at[b])
        pltpu.async_copy(mem_hbm.at[pl.ds(base + r, 1)], mem_v.at[b], m_sem.at[b])
        pltpu.async_copy(item_hbm.at[pl.ds(base + r, 1)], item_v.at[b], i_sem.at[b])

    for b in range(NBUF):
        issue_in(b, b)

    def step(g, carry):
        for b in range(NBUF):
            r = g * NBUF + b
            pltpu.make_async_copy(codes_hbm.at[pl.ds(0, 1)], code_v.at[b], c_sem.at[b]).wait()
            pltpu.make_async_copy(mem_hbm.at[pl.ds(0, 1)], mem_v.at[b], m_sem.at[b]).wait()
            pltpu.make_async_copy(item_hbm.at[pl.ds(0, 1)], item_v.at[b], i_sem.at[b]).wait()

            @pl.when(g > 0)
            def _wait_out():
                pltpu.make_async_copy(out_v.at[b], out_hbm.at[pl.ds(0, 1)], o_sem.at[b]).wait()

            gv = gate_v[pl.ds(pl.multiple_of(r * L, L), L)]
            one_m_g = 1.0 - gv

            def col(j, c2):
                loaded = []
                for u in range(UNROLL):
                    cs = pl.ds(j * (L * UNROLL) + u * L, L)
                    loaded.append((mem_v.at[b][0, cs], item_v.at[b][0, cs],
                                   code_v.at[b][0, cs]))
                for u in range(UNROLL):
                    cs = pl.ds(j * (L * UNROLL) + u * L, L)
                    m, it, cd = loaded[u]
                    out_v.at[b][0, cs] = gv * m + one_m_g * (it * cd)
                return c2

            lax.fori_loop(0, D // L // UNROLL, col, 0)

            pltpu.async_copy(out_v.at[b], out_hbm.at[pl.ds(base + r, 1)], o_sem.at[b])

            @pl.when(r + NBUF < BPW)
            def _prefetch():
                issue_in(r + NBUF, b)
        return carry

    lax.fori_loop(0, BPW // NBUF, step, 0)

    for b in range(NBUF):
        pltpu.make_async_copy(out_v.at[b], out_hbm.at[pl.ds(0, 1)], o_sem.at[b]).wait()


@jax.jit
def kernel(memory, item, position, gate, position_codes):
    pos8 = jnp.repeat(position.astype(jnp.int32), 8)
    gate16 = jnp.repeat(gate.reshape(-1), L)
    codes_pad = jnp.pad(position_codes, ((0, 0), (0, DPAD - D)))
    f = pl.kernel(
        _body,
        out_type=jax.ShapeDtypeStruct((B, D), jnp.float32),
        mesh=plsc.VectorSubcoreMesh(core_axis_name="c", subcore_axis_name="s"),
        compiler_params=pltpu.CompilerParams(use_tc_tiling_on_sc=True),
        scratch_types=[
            pltpu.VMEM((BPW * 8,), jnp.int32),
            pltpu.VMEM((BPW * L,), jnp.float32),
            pltpu.VMEM((NBUF, 1, DPAD), jnp.float32),
            pltpu.VMEM((NBUF, 1, D), jnp.float32),
            pltpu.VMEM((NBUF, 1, D), jnp.float32),
            pltpu.VMEM((NBUF, 1, D), jnp.float32),
            pltpu.SemaphoreType.DMA((NBUF,)),
            pltpu.SemaphoreType.DMA((NBUF,)),
            pltpu.SemaphoreType.DMA((NBUF,)),
            pltpu.SemaphoreType.DMA((NBUF,)),
        ],
    )
    return f(memory, item, pos8, gate16, codes_pad)


# speedup vs baseline: 6.9247x; 4.5556x over previous
import functools

import jax
import jax.numpy as jnp
from jax import lax
from jax.experimental import pallas as pl
from jax.experimental.pallas import tpu as pltpu

N_CB = 256
D = 10000
B = 4096
BD = 200


def _body(pos_ref, gate_ref, codes_ref, mem_ref, item_ref, out_ref, oh_ref):
    i = pl.program_id(0)

    @pl.when(i == 0)
    def _build_onehot():
        idx = lax.rem(pos_ref[...], jnp.full_like(pos_ref[...], N_CB))
        rows = lax.broadcasted_iota(jnp.int32, (N_CB, B), 0)
        oh_ref[...] = (rows == idx).astype(jnp.bfloat16)

    cd = jax.lax.dot_general(
        codes_ref[...].astype(jnp.bfloat16), oh_ref[...],
        (((1,), (0,)), ((), ())),
        preferred_element_type=jnp.float32)
    g = gate_ref[...]
    out_ref[...] = g * mem_ref[...] + (1.0 - g) * (item_ref[...] * cd)


@jax.jit
def kernel(memory, item, position, gate, position_codes):
    pos = position.astype(jnp.int32).reshape(1, B)
    gate_row = gate.reshape(1, B)
    mem_t = memory.T
    item_t = item.T
    codes_t = position_codes.T

    grid = D // BD
    out_t = pl.pallas_call(
        _body,
        grid=(grid,),
        in_specs=[
            pl.BlockSpec((1, B), lambda i: (0, 0)),
            pl.BlockSpec((1, B), lambda i: (0, 0)),
            pl.BlockSpec((BD, N_CB), lambda i: (i, 0)),
            pl.BlockSpec((BD, B), lambda i: (i, 0)),
            pl.BlockSpec((BD, B), lambda i: (i, 0)),
        ],
        out_specs=pl.BlockSpec((BD, B), lambda i: (i, 0)),
        out_shape=jax.ShapeDtypeStruct((D, B), jnp.float32),
        scratch_shapes=[pltpu.VMEM((N_CB, B), jnp.bfloat16)],
        compiler_params=pltpu.CompilerParams(
            dimension_semantics=("arbitrary",)),
    )(pos, gate_row, codes_t, mem_t, item_t)
    return out_t.T


# speedup vs baseline: 7.0084x; 1.0121x over previous
import jax
import jax.numpy as jnp
from jax import lax
from jax.experimental import pallas as pl
from jax.experimental.pallas import tpu as pltpu

N_CB = 256
D = 10000
B = 4096
BD = 400


def _body(pos_ref, gate_ref, codes_ref, mem_ref, item_ref, out_ref, oh_ref):
    i = pl.program_id(0)

    @pl.when(i == 0)
    def _build_onehot():
        idx = lax.rem(pos_ref[...], jnp.full_like(pos_ref[...], N_CB))
        rows = lax.broadcasted_iota(jnp.int32, (N_CB, B), 0)
        oh_ref[...] = (rows == idx).astype(jnp.bfloat16)

    cd = jax.lax.dot_general(
        codes_ref[...].astype(jnp.bfloat16), oh_ref[...],
        (((1,), (0,)), ((), ())),
        preferred_element_type=jnp.float32)
    g = gate_ref[...]
    out_ref[...] = g * mem_ref[...] + (1.0 - g) * (item_ref[...] * cd)


@jax.jit
def kernel(memory, item, position, gate, position_codes):
    pos = position.astype(jnp.int32).reshape(1, B)
    gate_row = gate.reshape(1, B)
    mem_t = memory.T
    item_t = item.T
    codes_t = position_codes.T

    grid = D // BD
    out_t = pl.pallas_call(
        _body,
        grid=(grid,),
        in_specs=[
            pl.BlockSpec((1, B), lambda i: (0, 0)),
            pl.BlockSpec((1, B), lambda i: (0, 0)),
            pl.BlockSpec((BD, N_CB), lambda i: (i, 0)),
            pl.BlockSpec((BD, B), lambda i: (i, 0)),
            pl.BlockSpec((BD, B), lambda i: (i, 0)),
        ],
        out_specs=pl.BlockSpec((BD, B), lambda i: (i, 0)),
        out_shape=jax.ShapeDtypeStruct((D, B), jnp.float32),
        scratch_shapes=[pltpu.VMEM((N_CB, B), jnp.bfloat16)],
        compiler_params=pltpu.CompilerParams(
            dimension_semantics=("arbitrary",)),
    )(pos, gate_row, codes_t, mem_t, item_t)
    return out_t.T
